# Initial kernel scaffold; baseline (speedup 1.0000x reference)
import jax
import jax.numpy as jnp
from jax.experimental import pallas as pl


def kernel(x, indices, values):
    raise NotImplementedError("write your pallas kernel here")


# speedup vs baseline: 1.4779x; 1.4779x over previous
import jax
import jax.numpy as jnp
from jax import lax
from jax.experimental import pallas as pl
from jax.experimental.pallas import tpu as pltpu
from jax.experimental.pallas import tpu_sc as plsc

_M = 1000000
_D = 64
_B = 16384
_NC = 2
_NS = 16
_NW = _NC * _NS
_RBIG = 31256
_RSML = 31232
_NBIG = 24
_C = 128
_L = 16
_WPAD = ((_RBIG + _L - 1) // _L) * _L
_CB = 256
_NFULL = _RSML // _CB
_TAIL = _RBIG - _NFULL * _CB


def _body(x_hbm, idx_hbm, val_hbm, out_hbm,
          idx_v, wtab, hits, cpos, cidx, rows, cbuf0, cbuf1,
          rsem0, rsem1, wsem0, wsem1, gsem, ssem):
    wid = lax.axis_index("s") * _NC + lax.axis_index("c")
    big = wid < _NBIG
    lo = jnp.where(big, wid * _RBIG, _NBIG * _RBIG + (wid - _NBIG) * _RSML)
    hi = lo + jnp.where(big, _RBIG, _RSML)

    pltpu.sync_copy(idx_hbm, idx_v)

    iota = lax.iota(jnp.int32, _L)
    neg1 = jnp.full((_L,), -1, jnp.int32)

    def fi(j, u):
        wtab[pl.ds(j * _L, _L)] = neg1
        return u

    lax.fori_loop(0, _WPAD // _L, fi, jnp.int32(0))

    def fc(j, cur):
        v = idx_v[pl.ds(j * _L, _L)]
        m = (v >= lo) & (v < hi)
        mi = m.astype(jnp.int32)
        offs = plsc.cumsum(mi) - mi
        plsc.store_scatter(hits, [cur + offs], iota + j * _L, mask=m)
        return cur + jnp.sum(mi)

    h = lax.fori_loop(0, _B // _L, fc, jnp.int32(0))

    def fd(j, u):
        valid = (iota + j * _L) < h
        p = jnp.where(valid, hits[pl.ds(j * _L, _L)], 0)
        m = plsc.load_gather(idx_v, [p]) - lo
        m = jnp.where(valid, m, 0)
        plsc.store_scatter(wtab, [m], p, mask=valid)

        def cond(w):
            return jnp.any(valid & (w < p))

        def body(w):
            plsc.store_scatter(wtab, [m], p, mask=valid & (w < p))
            return plsc.load_gather(wtab, [m])

        lax.while_loop(cond, body, plsc.load_gather(wtab, [m]))
        return u

    lax.fori_loop(0, (h + _L - 1) >> 4, fd, jnp.int32(0))

    def fe(j, wcur):
        valid = (iota + j * _L) < h
        p = jnp.where(valid, hits[pl.ds(j * _L, _L)], 0)
        m = plsc.load_gather(idx_v, [p]) - lo
        m = jnp.where(valid, m, 0)
        w = plsc.load_gather(wtab, [m])
        win = (w == p) & valid
        wi = win.astype(jnp.int32)
        offs = plsc.cumsum(wi) - wi
        plsc.store_scatter(hits, [wcur + offs], p, mask=win)
        return wcur + jnp.sum(wi)

    nw = lax.fori_loop(0, (h + _L - 1) >> 4, fe, jnp.int32(0))

    bufs = (cbuf0, cbuf1)
    rsems = (rsem0, rsem1)
    wsems = (wsem0, wsem1)

    def fcopy(g, u):
        for b in range(2):
            c = 2 * g + b
            off = lo + c * _CB

            @pl.when(c >= 2)
            def _():
                pltpu.make_async_copy(
                    bufs[b], out_hbm.at[pl.ds(off - 2 * _CB, _CB)],
                    wsems[b]).wait()

            rd = pltpu.make_async_copy(
                x_hbm.at[pl.ds(off, _CB)], bufs[b], rsems[b])
            rd.start()
            rd.wait()
            pltpu.make_async_copy(
                bufs[b], out_hbm.at[pl.ds(off, _CB)], wsems[b]).start()
        return u

    lax.fori_loop(0, _NFULL // 2, fcopy, jnp.int32(0))
    pltpu.make_async_copy(
        bufs[0], out_hbm.at[pl.ds(lo, _CB)], wsems[0]).wait()
    pltpu.make_async_copy(
        bufs[1], out_hbm.at[pl.ds(lo, _CB)], wsems[1]).wait()

    @pl.when(big)
    def _():
        toff = lo + _NFULL * _CB
        rd = pltpu.make_async_copy(
            x_hbm.at[pl.ds(toff, _TAIL)], rows.at[pl.ds(0, _TAIL)], rsem0)
        rd.start()
        rd.wait()
        wr = pltpu.make_async_copy(
            rows.at[pl.ds(0, _TAIL)], out_hbm.at[pl.ds(toff, _TAIL)], wsem0)
        wr.start()
        wr.wait()

    @pl.when(nw > 0)
    def _():
        p0 = jnp.broadcast_to(hits[pl.ds(0, _L)][0], (_L,))
        nchunks = (nw + _C - 1) >> 7
        npad = nchunks << 7

        for k in range(_C // _L):
            @pl.when(nw + k * _L < npad)
            def _():
                hits[pl.ds(nw + k * _L, _L)] = p0

        def fg(c, u):
            base = c << 7

            def fb(q, u2):
                pv = hits[pl.ds(base + q * _L, _L)]
                cpos[pl.ds(q * _L, _L)] = pv
                cidx[pl.ds(q * _L, _L)] = plsc.load_gather(idx_v, [pv])
                return u2

            lax.fori_loop(0, _C // _L, fb, jnp.int32(0))
            pltpu.async_copy(val_hbm.at[cpos], rows, gsem).wait()
            pltpu.async_copy(rows, out_hbm.at[cidx], ssem).wait()
            return u

        lax.fori_loop(0, nchunks, fg, jnp.int32(0))


@jax.jit
def kernel(x, indices, values):
    mesh = plsc.VectorSubcoreMesh(core_axis_name="c", subcore_axis_name="s")
    k = pl.kernel(
        _body,
        out_type=jax.ShapeDtypeStruct((_M, _D), jnp.float32),
        mesh=mesh,
        compiler_params=pltpu.CompilerParams(
            needs_layout_passes=False, use_tc_tiling_on_sc=False),
        scratch_types=[
            pltpu.VMEM((_B,), jnp.int32),
            pltpu.VMEM((_WPAD,), jnp.int32),
            pltpu.VMEM((_B + _L,), jnp.int32),
            pltpu.VMEM((_C,), jnp.int32),
            pltpu.VMEM((_C,), jnp.int32),
            pltpu.VMEM((_C, _D), jnp.float32),
            pltpu.VMEM((_CB, _D), jnp.float32),
            pltpu.VMEM((_CB, _D), jnp.float32),
            pltpu.SemaphoreType.DMA,
            pltpu.SemaphoreType.DMA,
            pltpu.SemaphoreType.DMA,
            pltpu.SemaphoreType.DMA,
            pltpu.SemaphoreType.DMA,
            pltpu.SemaphoreType.DMA,
        ],
    )
    return k(x, indices.reshape(_B), values)


# speedup vs baseline: 3.9654x; 2.6831x over previous
import jax
import jax.numpy as jnp
from jax import lax
from jax.experimental import pallas as pl
from jax.experimental.pallas import tpu as pltpu
from jax.experimental.pallas import tpu_sc as plsc

_M = 1000000
_D = 64
_B = 16384
_NC = 2
_NS = 16
_NW = _NC * _NS
_RW = 31232
_RLAST = _M - 31 * _RW
_L = 16
_CBC = 256
_NP0 = _RW // _CBC // 2
_NP1 = _RLAST // _CBC // 2
_TAIL = _RLAST - 2 * _NP1 * _CBC
_WTN = _RLAST


def _body(xt_hbm, idx_hbm, v2_hbm, out_hbm,
          idx_v, wtab, hits, clist, cpos, rows, cbuf0, cbuf1, tbuf,
          rsem0, rsem1, wsem0, wsem1, gsem):
    wid = lax.axis_index("s") * _NC + lax.axis_index("c")
    last = wid == _NW - 1
    lo = wid * _RW
    hi = lo + jnp.where(last, _RLAST, _RW)

    pltpu.sync_copy(idx_hbm, idx_v)

    iota = lax.iota(jnp.int32, _L)
    neg1 = jnp.full((_L,), -1, jnp.int32)

    def fi(j, u):
        wtab[pl.ds(j * _L, _L)] = neg1
        return u

    lax.fori_loop(0, _WTN // _L, fi, jnp.int32(0))

    def fc(j, cur):
        v = idx_v[pl.ds(j * _L, _L)]
        m = (v >= lo) & (v < hi)
        mi = m.astype(jnp.int32)
        offs = plsc.cumsum(mi) - mi
        plsc.store_scatter(hits, [cur + offs], iota + j * _L, mask=m)
        return cur + jnp.sum(mi)

    h = lax.fori_loop(0, _B // _L, fc, jnp.int32(0))

    def fd(j, u):
        valid = (iota + j * _L) < h
        p = jnp.where(valid, hits[pl.ds(j * _L, _L)], 0)
        m = plsc.load_gather(idx_v, [p]) - lo
        m = jnp.where(valid, m, 0)
        plsc.store_scatter(wtab, [m], p, mask=valid)

        def cond(w):
            return jnp.any(valid & (w < p))

        def bodyw(w):
            plsc.store_scatter(wtab, [m], p, mask=valid & (w < p))
            return plsc.load_gather(wtab, [m])

        lax.while_loop(cond, bodyw, plsc.load_gather(wtab, [m]))
        return u

    lax.fori_loop(0, (h + _L - 1) >> 4, fd, jnp.int32(0))

    def fe(j, wcur):
        valid = (iota + j * _L) < h
        p = jnp.where(valid, hits[pl.ds(j * _L, _L)], 0)
        m = plsc.load_gather(idx_v, [p]) - lo
        m = jnp.where(valid, m, 0)
        w = plsc.load_gather(wtab, [m])
        win = (w == p) & valid
        wi = win.astype(jnp.int32)
        offs = plsc.cumsum(wi) - wi
        plsc.store_scatter(hits, [wcur + offs], p, mask=win)
        return wcur + jnp.sum(wi)

    nw = lax.fori_loop(0, (h + _L - 1) >> 4, fe, jnp.int32(0))
    nwv = (nw + _L - 1) >> 4

    def scan_window(c0, span):
        def fs(j, cc):
            valid = (iota + j * _L) < nw
            p = jnp.where(valid, hits[pl.ds(j * _L, _L)], 0)
            m = plsc.load_gather(idx_v, [p])
            inw = valid & (m >= c0) & (m < c0 + span)
            ii = inw.astype(jnp.int32)
            offs = plsc.cumsum(ii) - ii
            plsc.store_scatter(clist, [cc + offs], p, mask=inw)
            return cc + jnp.sum(ii)

        return lax.fori_loop(0, nwv, fs, jnp.int32(0))

    def patch(buf, c0, ccount):
        @pl.when(ccount > 0)
        def _():
            p0 = jnp.broadcast_to(clist[pl.ds(0, _L)][0], (_L,))
            clist[pl.ds(ccount, _L)] = p0

            nbat = (ccount + 127) >> 7

            def fb(bi, u):
                bstart = bi << 7
                bcnt = jnp.minimum(ccount - bstart, 128)
                units = (bcnt + _L - 1) >> 4

                def fu(uu, u2):
                    pv = clist[pl.ds(bstart + uu * _L, _L)]
                    cpos[pl.ds(uu * _L, _L)] = pv >> 1
                    return u2

                lax.fori_loop(0, units, fu, jnp.int32(0))

                def fg(uu, u2):
                    pltpu.async_copy(
                        v2_hbm.at[cpos.at[pl.ds(uu * _L, _L)]],
                        rows.at[pl.ds(uu * _L, _L)], gsem).wait()
                    return u2

                lax.fori_loop(0, units, fg, jnp.int32(0))

                def fp(g, u2):
                    pv = clist[pl.ds(bstart + g * _L, _L)]
                    mloc = plsc.load_gather(idx_v, [pv]) - c0
                    par = (pv & 1) << 6
                    jvec = iota + g * _L

                    def fr(r, u3):
                        vals = plsc.load_gather(rows, [jvec, par + r])
                        rv = jnp.broadcast_to(r, (_L,))
                        plsc.store_scatter(buf, [rv, mloc], vals)
                        return u3

                    lax.fori_loop(0, _D, fr, jnp.int32(0))
                    return u2

                lax.fori_loop(0, units, fp, jnp.int32(0))
                return u

            lax.fori_loop(0, nbat, fb, jnp.int32(0))

    bufs = (cbuf0, cbuf1)
    rsems = (rsem0, rsem1)
    wsems = (wsem0, wsem1)
    npairs = jnp.where(last, _NP1, _NP0)

    def fpair(g, u):
        for b in range(2):
            c = 2 * g + b
            c0 = lo + c * _CBC

            @pl.when(c >= 2)
            def _():
                pltpu.make_async_copy(
                    bufs[b], out_hbm.at[:, pl.ds(c0 - 2 * _CBC, _CBC)],
                    wsems[b]).wait()

            rd = pltpu.make_async_copy(
                xt_hbm.at[:, pl.ds(c0, _CBC)], bufs[b], rsems[b])
            rd.start()
            ccount = scan_window(c0, _CBC)
            rd.wait()
            patch(bufs[b], c0, ccount)
            pltpu.make_async_copy(
                bufs[b], out_hbm.at[:, pl.ds(c0, _CBC)], wsems[b]).start()
        return u

    lax.fori_loop(0, npairs, fpair, jnp.int32(0))
    pltpu.make_async_copy(
        bufs[0], out_hbm.at[:, pl.ds(lo, _CBC)], wsems[0]).wait()
    pltpu.make_async_copy(
        bufs[1], out_hbm.at[:, pl.ds(lo, _CBC)], wsems[1]).wait()

    @pl.when(last)
    def _():
        c0 = _M - _TAIL
        rd = pltpu.make_async_copy(
            xt_hbm.at[:, pl.ds(c0, _TAIL)], tbuf, rsem0)
        rd.start()
        ccount = scan_window(c0, _TAIL)
        rd.wait()
        patch(tbuf, c0, ccount)
        wr = pltpu.make_async_copy(
            tbuf, out_hbm.at[:, pl.ds(c0, _TAIL)], wsem0)
        wr.start()
        wr.wait()


@jax.jit
def kernel(x, indices, values):
    mesh = plsc.VectorSubcoreMesh(core_axis_name="c", subcore_axis_name="s")
    k = pl.kernel(
        _body,
        out_type=jax.ShapeDtypeStruct((_D, _M), jnp.float32),
        mesh=mesh,
        compiler_params=pltpu.CompilerParams(needs_layout_passes=False),
        scratch_types=[
            pltpu.VMEM((_B,), jnp.int32),
            pltpu.VMEM((_WTN,), jnp.int32),
            pltpu.VMEM((_B + _L,), jnp.int32),
            pltpu.VMEM((_CBC + _L,), jnp.int32),
            pltpu.VMEM((128,), jnp.int32),
            pltpu.VMEM((128, 128), jnp.float32),
            pltpu.VMEM((_D, _CBC), jnp.float32),
            pltpu.VMEM((_D, _CBC), jnp.float32),
            pltpu.VMEM((_D, _TAIL), jnp.float32),
            pltpu.SemaphoreType.DMA,
            pltpu.SemaphoreType.DMA,
            pltpu.SemaphoreType.DMA,
            pltpu.SemaphoreType.DMA,
            pltpu.SemaphoreType.DMA,
        ],
    )
    outt = k(x.T, indices.reshape(_B), values.reshape(_B // 2, 128))
    return outt.T
